# baseline (speedup 1.0000x reference)
import functools

import jax
import jax.numpy as jnp
from jax import lax
from jax.experimental import pallas as pl
from jax.experimental.pallas import tpu as pltpu
from jax.experimental.pallas import tpu_sc as plsc

_B = 512
_NPIX = 3 * 224 * 224
_NRET = 128
_NCLS = 1000
_NC, _NS, _L = 2, 16, 16
_NG = 16
_QB = 128


def _sigmoid(z):
    return 1.0 / (1.0 + jnp.exp(-z))


def _retina_sc(xt, aux):
    mesh = plsc.VectorSubcoreMesh(core_axis_name="c", subcore_axis_name="s")

    @functools.partial(
        pl.kernel,
        out_type=jax.ShapeDtypeStruct((_NRET, _B), jnp.float32),
        mesh=mesh,
        scratch_types=[
            pltpu.VMEM((4224,), jnp.int32),
            pltpu.VMEM((_NG * 16, _QB), jnp.float32),
            pltpu.VMEM((_NG, _QB), jnp.float32),
            pltpu.SemaphoreType.DMA,
            pltpu.SemaphoreType.DMA,
            pltpu.SemaphoreType.DMA,
            pltpu.SemaphoreType.DMA,
        ],
        compiler_params=pltpu.CompilerParams(needs_layout_passes=False),
    )
    def k(xt_h, aux_h, out_h, aux_v, stage, r1l, sem0, sem1, sem2, sem3):
        cid = lax.axis_index("c")
        sid = lax.axis_index("s")
        wid = sid * _NC + cid
        ng = lax.shift_right_logical(wid, 2)
        q = lax.bitwise_and(wid, 3)
        n0 = pl.multiple_of(ng * _NG, _NG)
        col0 = pl.multiple_of(q * _QB, _QB)

        p0 = n0 * 16
        pltpu.sync_copy(aux_h.at[pl.ds(p0, 2 * _QB)],
                        aux_v.at[pl.ds(p0, 2 * _QB)])
        sems = (sem0, sem1, sem2, sem3)
        _GR = 64
        for g in range(4):
            pltpu.async_copy(
                xt_h.at[aux_v.at[pl.ds(p0 + g * _GR, _GR)], pl.ds(col0, _QB)],
                stage.at[pl.ds(g * _GR, _GR)], sems[g])
        pltpu.sync_copy(aux_h.at[pl.ds(2048, 2176)],
                        aux_v.at[pl.ds(2048, 2176)])

        def compute(nl, carry):
            n = n0 + nl
            acc0 = plsc.bitcast(
                plsc.load_gather(aux_v, [jnp.broadcast_to(4096 + n, (_L,))]),
                jnp.float32)
            wsp = [plsc.bitcast(
                plsc.load_gather(
                    aux_v, [jnp.broadcast_to(2048 + n * 16 + j, (_L,))]),
                jnp.float32) for j in range(16)]
            for bb in range(_QB // _L):
                acc = acc0
                for j in range(16):
                    acc = acc + (stage[nl * 16 + j, pl.ds(bb * _L, _L)]
                                 * wsp[j])
                r1l[nl, pl.ds(bb * _L, _L)] = _sigmoid(acc)
            return carry

        for g in range(4):
            pltpu.make_async_copy(
                xt_h.at[aux_v.at[pl.ds(p0 + g * _GR, _GR)], pl.ds(col0, _QB)],
                stage.at[pl.ds(g * _GR, _GR)], sems[g]).wait()
            lax.fori_loop(g * (_NG // 4), (g + 1) * (_NG // 4), compute, 0)

        pltpu.sync_copy(r1l, out_h.at[pl.ds(n0, _NG), pl.ds(col0, _QB)])

    return k(xt, aux)


def _dense_tc_body(r1_ref, wl_ref, bl_ref, m1_ref, wv_ref, bv_ref,
                   m2_ref, wi_ref, bi_ref, wc_ref, bc_ref, out_ref):
    f32 = jnp.float32
    hi = lax.Precision.DEFAULT
    r1t = r1_ref[...]
    wl = jnp.sum(wl_ref[...].reshape(_NRET, 16), axis=1)
    r2t = _sigmoid(r1t * wl[:, None] + bl_ref[...][:, None])
    m1 = m1_ref[...]
    deg1 = jnp.sum(m1, axis=1)
    mv1t = lax.dot_general(m1, r2t, (((1,), (0,)), ((), ())),
                           precision=hi, preferred_element_type=f32)
    mv1t = mv1t / deg1[:, None]
    wv = jnp.sum(wv_ref[...].reshape(256, 32), axis=1)
    r3t = _sigmoid(mv1t * wv[:, None] + bv_ref[...][:, None])
    m2 = m2_ref[...]
    deg2 = jnp.sum(m2, axis=1)
    mitt = lax.dot_general(m2, r3t, (((1,), (0,)), ((), ())),
                           precision=hi, preferred_element_type=f32)
    mitt = mitt / deg2[:, None]
    wi = jnp.sum(wi_ref[...].reshape(_NRET, 32), axis=1)
    r4t = _sigmoid(mitt * wi[:, None] + bi_ref[...][:, None])
    out_ref[...] = (lax.dot_general(wc_ref[...], r4t, (((1,), (0,)), ((), ())),
                                    precision=hi, preferred_element_type=f32)
                    + bc_ref[...][:, None])


def _dense_tc(r1t, W_lgn, b_lgn, m1, W_v1, b_v1, m2, W_it, b_it, W_cls, b_cls):
    return pl.pallas_call(
        _dense_tc_body,
        out_shape=jax.ShapeDtypeStruct((_NCLS, _B), jnp.float32),
        compiler_params=pltpu.CompilerParams(skip_device_barrier=True),
    )(r1t, W_lgn, b_lgn, m1, W_v1, b_v1, m2, W_it, b_it, W_cls, b_cls)


def kernel(x, W_ret, b_ret, W_lgn, b_lgn, W_v1, b_v1, W_it, b_it,
           W_cls, b_cls, pixel_map, lgn_to_v1, v1_to_it):
    xt = x.reshape(_B, _NPIX).T
    aux = jnp.concatenate([
        pixel_map.reshape(-1).astype(jnp.int32),
        lax.bitcast_convert_type(W_ret.reshape(-1), jnp.int32),
        lax.bitcast_convert_type(b_ret, jnp.int32),
    ])
    r1t = _retina_sc(xt, aux)
    logits_t = _dense_tc(r1t, W_lgn, b_lgn, lgn_to_v1, W_v1, b_v1,
                         v1_to_it, W_it, b_it, W_cls, b_cls)
    return logits_t.T


# speedup vs baseline: 1.0439x; 1.0439x over previous
import functools

import jax
import jax.numpy as jnp
from jax import lax
from jax.experimental import pallas as pl
from jax.experimental.pallas import tpu as pltpu
from jax.experimental.pallas import tpu_sc as plsc

_B = 512
_NPIX = 3 * 224 * 224
_NRET = 128
_NCLS = 1000
_NC, _NS, _L = 2, 16, 16
_NG = 16
_QB = 128


def _sigmoid(z):
    return 1.0 / (1.0 + jnp.exp(-z))


def _retina_sc(xt, aux):
    mesh = plsc.VectorSubcoreMesh(core_axis_name="c", subcore_axis_name="s")

    @functools.partial(
        pl.kernel,
        out_type=jax.ShapeDtypeStruct((_NRET, _B), jnp.float32),
        mesh=mesh,
        scratch_types=[
            pltpu.VMEM((4224,), jnp.int32),
            pltpu.VMEM((_NG * 16, _QB), jnp.float32),
            pltpu.VMEM((_NG, _QB), jnp.float32),
            pltpu.SemaphoreType.DMA,
            pltpu.SemaphoreType.DMA,
        ],
        compiler_params=pltpu.CompilerParams(needs_layout_passes=False),
    )
    def k(xt_h, aux_h, out_h, aux_v, stage, r1l, sem0, sem1):
        cid = lax.axis_index("c")
        sid = lax.axis_index("s")
        wid = sid * _NC + cid
        ng = lax.shift_right_logical(wid, 2)
        q = lax.bitwise_and(wid, 3)
        n0 = pl.multiple_of(ng * _NG, _NG)
        col0 = pl.multiple_of(q * _QB, _QB)

        p0 = n0 * 16
        pltpu.sync_copy(aux_h.at[pl.ds(p0, 2 * _QB)],
                        aux_v.at[pl.ds(p0, 2 * _QB)])
        sems = (sem0, sem1)
        for g in range(2):
            pltpu.async_copy(
                xt_h.at[aux_v.at[pl.ds(p0 + g * _QB, _QB)], pl.ds(col0, _QB)],
                stage.at[pl.ds(g * _QB, _QB)], sems[g])
        pltpu.sync_copy(aux_h.at[pl.ds(2048, 2176)],
                        aux_v.at[pl.ds(2048, 2176)])

        def compute(nl, carry):
            n = n0 + nl
            acc0 = plsc.bitcast(
                plsc.load_gather(aux_v, [jnp.broadcast_to(4096 + n, (_L,))]),
                jnp.float32)
            wsp = [plsc.bitcast(
                plsc.load_gather(
                    aux_v, [jnp.broadcast_to(2048 + n * 16 + j, (_L,))]),
                jnp.float32) for j in range(16)]
            for bb in range(_QB // _L):
                acc = acc0
                for j in range(16):
                    acc = acc + (stage[nl * 16 + j, pl.ds(bb * _L, _L)]
                                 * wsp[j])
                r1l[nl, pl.ds(bb * _L, _L)] = _sigmoid(acc)
            return carry

        for g in range(2):
            pltpu.make_async_copy(
                xt_h.at[aux_v.at[pl.ds(p0 + g * _QB, _QB)], pl.ds(col0, _QB)],
                stage.at[pl.ds(g * _QB, _QB)], sems[g]).wait()
            lax.fori_loop(g * (_NG // 2), (g + 1) * (_NG // 2), compute, 0)

        pltpu.sync_copy(r1l, out_h.at[pl.ds(n0, _NG), pl.ds(col0, _QB)])

    return k(xt, aux)


def _dense_tc_body(r1_ref, wl_ref, bl_ref, m1_ref, wv_ref, bv_ref,
                   m2_ref, wi_ref, bi_ref, wc_ref, bc_ref, out_ref):
    f32 = jnp.float32
    hi = lax.Precision.DEFAULT
    r1t = r1_ref[...]
    wl = jnp.sum(wl_ref[...].reshape(_NRET, 16), axis=1)
    r2t = _sigmoid(r1t * wl[:, None] + bl_ref[...][:, None])
    m1 = m1_ref[...]
    deg1 = jnp.sum(m1, axis=1)
    mv1t = lax.dot_general(m1, r2t, (((1,), (0,)), ((), ())),
                           precision=hi, preferred_element_type=f32)
    mv1t = mv1t / deg1[:, None]
    wv = jnp.sum(wv_ref[...].reshape(256, 32), axis=1)
    r3t = _sigmoid(mv1t * wv[:, None] + bv_ref[...][:, None])
    m2 = m2_ref[...]
    deg2 = jnp.sum(m2, axis=1)
    mitt = lax.dot_general(m2, r3t, (((1,), (0,)), ((), ())),
                           precision=hi, preferred_element_type=f32)
    mitt = mitt / deg2[:, None]
    wi = jnp.sum(wi_ref[...].reshape(_NRET, 32), axis=1)
    r4t = _sigmoid(mitt * wi[:, None] + bi_ref[...][:, None])
    wc_bf = wc_ref[...].astype(jnp.bfloat16)
    r4_bf = r4t.astype(jnp.bfloat16)
    out_ref[...] = (lax.dot_general(wc_bf, r4_bf, (((1,), (0,)), ((), ())),
                                    preferred_element_type=f32)
                    + bc_ref[...][:, None])


def _dense_tc(r1t, W_lgn, b_lgn, m1, W_v1, b_v1, m2, W_it, b_it, W_cls, b_cls):
    return pl.pallas_call(
        _dense_tc_body,
        out_shape=jax.ShapeDtypeStruct((_NCLS, _B), jnp.float32),
        compiler_params=pltpu.CompilerParams(skip_device_barrier=True),
    )(r1t, W_lgn, b_lgn, m1, W_v1, b_v1, m2, W_it, b_it, W_cls, b_cls)


def kernel(x, W_ret, b_ret, W_lgn, b_lgn, W_v1, b_v1, W_it, b_it,
           W_cls, b_cls, pixel_map, lgn_to_v1, v1_to_it):
    xt = x.reshape(_B, _NPIX).T
    aux = jnp.concatenate([
        pixel_map.reshape(-1).astype(jnp.int32),
        lax.bitcast_convert_type(W_ret.reshape(-1), jnp.int32),
        lax.bitcast_convert_type(b_ret, jnp.int32),
    ])
    r1t = _retina_sc(xt, aux)
    logits_t = _dense_tc(r1t, W_lgn, b_lgn, lgn_to_v1, W_v1, b_v1,
                         v1_to_it, W_it, b_it, W_cls, b_cls)
    return logits_t.T
